# baseline (speedup 1.0000x reference)
import jax
import jax.numpy as jnp
from jax import lax
from jax.experimental import pallas as pl
from jax.experimental.pallas import tpu as pltpu

B = 1024
N_PATCHES = 128
D_MODEL = 768
ROWS = B * N_PATCHES
CH = 512
NBUF = 8
NCH = ROWS // CH


def _body(rp_ref, w_ref, s_ref, out_hbm, buf, sems):
    i = pl.program_id(0)
    slot = lax.rem(i, NBUF)

    @pl.when(i >= NBUF)
    def _wait_slot():
        pltpu.make_async_copy(
            buf.at[slot], out_hbm.at[pl.ds(i * CH, CH), :], sems.at[slot]
        ).wait()

    buf[slot] = rp_ref[...] * (w_ref[...] * s_ref[0, 0])
    pltpu.make_async_copy(
        buf.at[slot], out_hbm.at[pl.ds(i * CH, CH), :], sems.at[slot]
    ).start()

    @pl.when(i == NCH - 1)
    def _drain():
        for k in range(NBUF):
            pltpu.make_async_copy(
                buf.at[k], out_hbm.at[pl.ds(0, CH), :], sems.at[k]
            ).wait()


def kernel(n_patches, relative_positions, W, scale):
    rp2 = relative_positions.reshape(ROWS, 1)
    w2 = W.reshape(1, D_MODEL)
    s2 = scale.reshape(1, 1)
    out = pl.pallas_call(
        _body,
        grid=(NCH,),
        in_specs=[
            pl.BlockSpec((CH, 1), lambda i: (i, 0)),
            pl.BlockSpec((1, D_MODEL), lambda i: (0, 0)),
            pl.BlockSpec((1, 1), lambda i: (0, 0)),
        ],
        out_specs=pl.BlockSpec(memory_space=pl.ANY),
        out_shape=jax.ShapeDtypeStruct((ROWS, D_MODEL), jnp.float32),
        scratch_shapes=[
            pltpu.VMEM((NBUF, CH, D_MODEL), jnp.float32),
            pltpu.SemaphoreType.DMA((NBUF,)),
        ],
    )(rp2, w2, s2)
    return out.reshape(B, N_PATCHES, D_MODEL)


# speedup vs baseline: 1.3684x; 1.3684x over previous
import jax
import jax.numpy as jnp
from jax.experimental import pallas as pl

B = 1024
N_PATCHES = 128
D_MODEL = 768
ROW_BLOCK = 2048


def _body(rp_ref, w_ref, s_ref, out_ref):
    out_ref[...] = rp_ref[...] * (w_ref[...] * s_ref[0, 0])


def kernel(n_patches, relative_positions, W, scale):
    rows = B * N_PATCHES
    rp2 = relative_positions.reshape(rows, 1)
    w2 = W.reshape(1, D_MODEL)
    s2 = scale.reshape(1, 1)
    grid = (rows // ROW_BLOCK,)
    out = pl.pallas_call(
        _body,
        grid=grid,
        in_specs=[
            pl.BlockSpec((ROW_BLOCK, 1), lambda i: (i, 0)),
            pl.BlockSpec((1, D_MODEL), lambda i: (0, 0)),
            pl.BlockSpec((1, 1), lambda i: (0, 0)),
        ],
        out_specs=pl.BlockSpec((ROW_BLOCK, D_MODEL), lambda i: (i, 0)),
        out_shape=jax.ShapeDtypeStruct((rows, D_MODEL), jnp.float32),
    )(rp2, w2, s2)
    return out.reshape(B, N_PATCHES, D_MODEL)


# speedup vs baseline: 2.0279x; 1.4819x over previous
import jax
import jax.numpy as jnp
from jax.experimental import pallas as pl

B = 1024
N_PATCHES = 128
D_MODEL = 768
BB = 16


def _body(rp_ref, w_ref, s_ref, out_ref):
    wv = (w_ref[...] * s_ref[0, 0]).reshape(1, 1, D_MODEL)
    out_ref[...] = rp_ref[...][:, :, None] * wv


def kernel(n_patches, relative_positions, W, scale):
    w2 = W.reshape(1, D_MODEL)
    s2 = scale.reshape(1, 1)
    grid = (B // BB,)
    out = pl.pallas_call(
        _body,
        grid=grid,
        in_specs=[
            pl.BlockSpec((BB, N_PATCHES), lambda i: (i, 0)),
            pl.BlockSpec((1, D_MODEL), lambda i: (0, 0)),
            pl.BlockSpec((1, 1), lambda i: (0, 0)),
        ],
        out_specs=pl.BlockSpec((BB, N_PATCHES, D_MODEL), lambda i: (i, 0, 0)),
        out_shape=jax.ShapeDtypeStruct((B, N_PATCHES, D_MODEL), jnp.float32),
    )(relative_positions, w2, s2)
    return out
